# baseline (speedup 1.0000x reference)
import jax
import jax.numpy as jnp
from jax import lax
from jax.experimental import pallas as pl
from jax.experimental.pallas import tpu as pltpu

N, D, T = 100000, 32, 8
N4 = N // 4
B4 = 2048
NB = -(-N4 // B4)
R4 = NB * B4
CROWS = 256
CH = R4 // CROWS
PB = B4 // CROWS
VCH = N4 // CROWS
NEG = -1e30
BIGI = 2**31 - 1


def _body(tgt_sm, items4_blk, u_ref, items_any, out_ref, scr, p_ref, w, w4,
          sem):
    k = pl.program_id(0)

    @pl.when(k == 0)
    def _init():
        w[...] = jnp.zeros((16, D), jnp.float32)
        w[0:1, :] = u_ref[...]
        copies = []
        for i in range(T):
            c = pltpu.make_async_copy(
                items_any.at[pl.ds(tgt_sm[i], 1), :],
                w.at[pl.ds(1 + i, 1), :],
                sem,
            )
            c.start()
            copies.append(c)
        for c in copies:
            c.wait()
        w4[...] = jnp.zeros((128, 64), jnp.float32)
        wt = jnp.transpose(w[...])
        for j in range(4):
            w4[D * j:D * (j + 1), 16 * j:16 * (j + 1)] = wt

    x = items4_blk[...]
    s = lax.dot_general(x, w4[...], (((1,), (0,)), ((), ())),
                        preferred_element_type=jnp.float32)
    scr[pl.ds(k * B4, B4), :] = s
    p_ref[pl.ds(k * PB, PB), :] = jnp.max(s.reshape(PB, CROWS, 64), axis=1)

    @pl.when(k == NB - 1)
    def _select():
        lane = lax.broadcasted_iota(jnp.int32, (1, 64), 1)
        chunk_iota = lax.broadcasted_iota(jnp.int32, (CH, 64), 0)
        rowi = lax.broadcasted_iota(jnp.int32, (CROWS, 64), 0)
        gloc = 4 * rowi + lax.broadcasted_iota(jnp.int32, (CROWS, 64), 1) // 16

        tail = scr[pl.ds(VCH * CROWS, CROWS), :]
        p_ref[VCH:VCH + 1, :] = jnp.max(
            jnp.where(VCH * CROWS + rowi < N4, tail, NEG), axis=0
        ).reshape(1, 64)
        p_ref[VCH + 1:, :] = jnp.full((CH - VCH - 1, 64), NEG, jnp.float32)

        def refresh_p(ci):
            sch = scr[pl.ds(ci * CROWS, CROWS), :]
            vrow = ci * CROWS + rowi < N4
            p_ref[pl.ds(ci, 1), :] = jnp.max(
                jnp.where(vrow, sch, NEG), axis=0).reshape(1, 64)

        def pick(c):
            sel = lane % 16 == c
            pm = jnp.where(sel, p_ref[...], NEG)
            m = jnp.max(pm)
            ci = jnp.min(jnp.where(pm == m, chunk_iota, BIGI))
            sch = scr[pl.ds(ci * CROWS, CROWS), :]
            vrow = ci * CROWS + rowi < N4
            hit = sel & vrow & (sch == m)
            g = ci * (4 * CROWS) + jnp.min(jnp.where(hit, gloc, BIGI))
            r = g // 4
            lidx = (g % 4) * 16 + c
            rowv = scr[pl.ds(r, 1), :]
            scr[pl.ds(r, 1), :] = jnp.where(lane == lidx, NEG, rowv)
            gl = g - ci * (4 * CROWS)
            sch2 = jnp.where(vrow & ~(sel & (gloc == gl)), sch, NEG)
            p_ref[pl.ds(ci, 1), :] = jnp.max(sch2, axis=0).reshape(1, 64)
            return g, m

        def exclude(c, g, cond=None):
            r = g // 4
            hit = lane == (g % 4) * 16 + c
            if cond is not None:
                hit = hit & cond
            rowv = scr[pl.ds(r, 1), :]
            scr[pl.ds(r, 1), :] = jnp.where(hit, NEG, rowv)
            refresh_p(r // CROWS)

        def score_at(g):
            rowv = scr[pl.ds(g // 4, 1), :]
            return jnp.sum(jnp.where(lane == (g % 4) * 16, rowv, 0.0))

        tops = []
        for _ in range(6):
            tops.append(pick(0))
        for g, m in tops:
            rowv = scr[pl.ds(g // 4, 1), :]
            scr[pl.ds(g // 4, 1), :] = jnp.where(lane == (g % 4) * 16, m,
                                                 rowv)

        loss = jnp.float32(0.0)
        for t in range(T):
            tt = tgt_sm[t]
            s_t = jnp.sum(w[0, :] * w[1 + t, :])

            in5 = tops[0][0] == tt
            for i in range(1, 5):
                in5 = in5 | (tops[i][0] == tt)
            contrib = jnp.float32(0.0)
            for i in range(5):
                contrib += jnp.where(tops[i][0] == tt, 0.0,
                                     jax.nn.sigmoid(tops[i][1] - s_t))
            contrib += jnp.where(in5, jax.nn.sigmoid(tops[5][1] - s_t), 0.0)

            c = 1 + t
            exclude(c, tt)
            for i in range(5):
                exclude(c, tops[i][0])
            exclude(c, tops[5][0], cond=in5)
            for _ in range(5):
                g, _m = pick(c)
                contrib += jax.nn.sigmoid(score_at(g) - s_t)

            loss += contrib
        out_ref[...] = jnp.broadcast_to(loss, (1, 1))


def kernel(items_emb, user_emb, target_items):
    items4 = items_emb.reshape(N4, 128)
    grid_spec = pltpu.PrefetchScalarGridSpec(
        num_scalar_prefetch=1,
        grid=(NB,),
        in_specs=[
            pl.BlockSpec((B4, 128), lambda k, tgt: (k, 0)),
            pl.BlockSpec((1, D), lambda k, tgt: (0, 0)),
            pl.BlockSpec(memory_space=pltpu.MemorySpace.HBM),
        ],
        out_specs=pl.BlockSpec((1, 1), lambda k, tgt: (0, 0)),
        scratch_shapes=[
            pltpu.VMEM((R4, 64), jnp.float32),
            pltpu.VMEM((CH, 64), jnp.float32),
            pltpu.VMEM((16, D), jnp.float32),
            pltpu.VMEM((128, 64), jnp.float32),
            pltpu.SemaphoreType.DMA,
        ],
    )
    out = pl.pallas_call(
        _body,
        grid_spec=grid_spec,
        out_shape=jax.ShapeDtypeStruct((1, 1), jnp.float32),
    )(target_items, items4, user_emb, items_emb)
    return out[0, 0]
